# Initial kernel scaffold; baseline (speedup 1.0000x reference)
import jax
import jax.numpy as jnp
from jax.experimental import pallas as pl


def kernel(x, edge_index, bn_gamma, bn_beta, W0, W1, cheb_b, mlp_bn1_g, mlp_bn1_b, mlp_W, mlp_b, mlp_bn2_g, mlp_bn2_b):
    raise NotImplementedError("write your pallas kernel here")


# speedup vs baseline: 7.1819x; 7.1819x over previous
import functools

import jax
import jax.numpy as jnp
from jax import lax
from jax.experimental import pallas as pl
from jax.experimental.pallas import tpu as pltpu
from jax.experimental.pallas import tpu_sc as plsc

NN = 10000
HH = 128
NCORES = 2
NSUB = 16
NW = NCORES * NSUB
CHUNK = 128
N_ACC = 10240
RPT = N_ACC // NSUB
G_PAD = NN + 16

_MESH = plsc.VectorSubcoreMesh(core_axis_name="c", subcore_axis_name="s")


def _deg_body(row_hbm, out_hbm, idx_v, ones_v, zb_v, acc):
    c = lax.axis_index("c")
    s = lax.axis_index("s")
    wid = s * NCORES + c
    cpw = row_hbm.shape[0] // (NW * CHUNK)
    for i in range(CHUNK // 16):
        ones_v[pl.ds(i * 16, 16)] = jnp.ones((16,), jnp.float32)

    def zfill(i, carry):
        zb_v[pl.ds(i * 16, 16)] = jnp.zeros((16,), jnp.float32)
        return carry

    lax.fori_loop(0, RPT // 16, zfill, 0)
    pltpu.sync_copy(zb_v, acc.at[pl.ds(s * RPT, RPT)])
    plsc.subcore_barrier()
    base = wid * cpw * CHUNK

    def body(j, carry):
        pltpu.sync_copy(row_hbm.at[pl.ds(base + j * CHUNK, CHUNK)], idx_v)
        pltpu.sync_copy(ones_v, acc.at[idx_v], add=True)
        return carry

    lax.fori_loop(0, cpw, body, 0)
    plsc.subcore_barrier()
    pltpu.sync_copy(acc.at[pl.ds(s * RPT, RPT)],
                    out_hbm.at[c, pl.ds(s * RPT, RPT)])


def _seg_body(g_hbm, row_hbm, col_hbm, out_hbm, idxr_v, idxc_v, rows_v, zb_v, acc):
    c = lax.axis_index("c")
    s = lax.axis_index("s")
    wid = s * NCORES + c
    cpw = row_hbm.shape[0] // (NW * CHUNK)
    for i in range(16):
        for k in range(HH // 16):
            zb_v[i, pl.ds(k * 16, 16)] = jnp.zeros((16,), jnp.float32)

    def zcopy(i, carry):
        pltpu.sync_copy(zb_v, acc.at[pl.ds(s * RPT + i * 16, 16)])
        return carry

    lax.fori_loop(0, RPT // 16, zcopy, 0)
    plsc.subcore_barrier()
    base = wid * cpw * CHUNK

    def body(j, carry):
        off = base + j * CHUNK
        pltpu.sync_copy(row_hbm.at[pl.ds(off, CHUNK)], idxr_v)
        pltpu.sync_copy(col_hbm.at[pl.ds(off, CHUNK)], idxc_v)
        pltpu.sync_copy(g_hbm.at[idxr_v], rows_v)
        pltpu.sync_copy(rows_v, acc.at[idxc_v], add=True)
        return carry

    lax.fori_loop(0, cpw, body, 0)
    plsc.subcore_barrier()
    pltpu.sync_copy(acc.at[pl.ds(s * RPT, RPT)],
                    out_hbm.at[c, pl.ds(s * RPT, RPT)])


def _make_deg(e_pad):
    return pl.kernel(
        _deg_body,
        out_type=jax.ShapeDtypeStruct((NCORES, N_ACC), jnp.float32),
        mesh=_MESH,
        scratch_types=[
            pltpu.VMEM((CHUNK,), jnp.int32),
            pltpu.VMEM((CHUNK,), jnp.float32),
            pltpu.VMEM((RPT,), jnp.float32),
            pltpu.VMEM_SHARED((N_ACC,), jnp.float32),
        ],
    )


def _make_seg(e_pad):
    return pl.kernel(
        _seg_body,
        out_type=jax.ShapeDtypeStruct((NCORES, N_ACC, HH), jnp.float32),
        mesh=_MESH,
        scratch_types=[
            pltpu.VMEM((CHUNK,), jnp.int32),
            pltpu.VMEM((CHUNK,), jnp.int32),
            pltpu.VMEM((CHUNK, HH), jnp.float32),
            pltpu.VMEM((16, HH), jnp.float32),
            pltpu.VMEM_SHARED((N_ACC, HH), jnp.float32),
        ],
    )


def _bn_relu(x, g, b):
    m = jnp.mean(x, axis=0, keepdims=True)
    xc = x - m
    v = jnp.mean(xc * xc, axis=0, keepdims=True)
    return jnp.maximum(g * xc * lax.rsqrt(v + 1e-5) + b, 0.0)


def _pre_body(x_ref, deg_ref, gam_ref, bet_ref, h_ref, g_ref, dinv_ref):
    deg = deg_ref[...]
    dinv = jnp.where(deg > 0, lax.rsqrt(deg), 0.0)
    h = _bn_relu(x_ref[...], gam_ref[...], bet_ref[...])
    h_ref[...] = h
    g_ref[0:NN, :] = dinv * h
    g_ref[NN:G_PAD, :] = jnp.zeros((G_PAD - NN, HH), jnp.float32)
    dinv_ref[...] = dinv


def _dense_block(h, S_ref, dinv, res, W0_ref, W1_ref, cb_ref, g1_ref, b1_ref,
                 mW_ref, mb_ref, g2_ref, b2_ref):
    S = S_ref[0, 0:NN, :] + S_ref[1, 0:NN, :]
    Tx1 = -dinv * S
    out = (jnp.dot(h, W0_ref[...], preferred_element_type=jnp.float32)
           + jnp.dot(Tx1, W1_ref[...], preferred_element_type=jnp.float32)
           + cb_ref[...])
    h2 = _bn_relu(out, g1_ref[...], b1_ref[...])
    h3 = jnp.dot(h2, mW_ref[...], preferred_element_type=jnp.float32) + mb_ref[...]
    h4 = _bn_relu(h3, g2_ref[...], b2_ref[...])
    return h4 + res


def _post_fused_body(h_ref, S_ref, dinv_ref, res_ref, W0_ref, W1_ref, cb_ref,
                     g1_ref, b1_ref, mW_ref, mb_ref, g2_ref, b2_ref,
                     gn_ref, bnb_ref, x_ref, hn_ref, gn_out_ref):
    dinv = dinv_ref[...]
    xn = _dense_block(h_ref[...], S_ref, dinv, res_ref[...], W0_ref, W1_ref,
                      cb_ref, g1_ref, b1_ref, mW_ref, mb_ref, g2_ref, b2_ref)
    x_ref[...] = xn
    hn = _bn_relu(xn, gn_ref[...], bnb_ref[...])
    hn_ref[...] = hn
    gn_out_ref[0:NN, :] = dinv * hn
    gn_out_ref[NN:G_PAD, :] = jnp.zeros((G_PAD - NN, HH), jnp.float32)


def _post_final_body(h_ref, S_ref, dinv_ref, res_ref, W0_ref, W1_ref, cb_ref,
                     g1_ref, b1_ref, mW_ref, mb_ref, g2_ref, b2_ref, x_ref):
    dinv = dinv_ref[...]
    x_ref[...] = _dense_block(h_ref[...], S_ref, dinv, res_ref[...], W0_ref,
                              W1_ref, cb_ref, g1_ref, b1_ref, mW_ref, mb_ref,
                              g2_ref, b2_ref)


_f32 = jnp.float32
_pre_call = pl.pallas_call(
    _pre_body,
    out_shape=[jax.ShapeDtypeStruct((NN, HH), _f32),
               jax.ShapeDtypeStruct((G_PAD, HH), _f32),
               jax.ShapeDtypeStruct((NN, 1), _f32)],
)
_post_fused_call = pl.pallas_call(
    _post_fused_body,
    out_shape=[jax.ShapeDtypeStruct((NN, HH), _f32),
               jax.ShapeDtypeStruct((NN, HH), _f32),
               jax.ShapeDtypeStruct((G_PAD, HH), _f32)],
)
_post_final_call = pl.pallas_call(
    _post_final_body,
    out_shape=jax.ShapeDtypeStruct((NN, HH), _f32),
)


def kernel(x, edge_index, bn_gamma, bn_beta, W0, W1, cheb_b, mlp_bn1_g,
           mlp_bn1_b, mlp_W, mlp_b, mlp_bn2_g, mlp_bn2_b):
    row = edge_index[0]
    col = edge_index[1]
    e = row.shape[0]
    block = NW * CHUNK
    e_pad = ((e + block - 1) // block) * block
    pad = e_pad - e
    padv = jnp.full((pad,), NN, dtype=jnp.int32)
    row_p = jnp.concatenate([row, padv])
    col_p = jnp.concatenate([col, padv])

    deg_parts = _make_deg(e_pad)(row_p)
    deg = (deg_parts[0, :NN] + deg_parts[1, :NN]).reshape(NN, 1)

    seg = _make_seg(e_pad)

    h, g, dinv = _pre_call(x, deg, bn_gamma[0].reshape(1, HH),
                           bn_beta[0].reshape(1, HH))
    for i in range(3):
        S = seg(g, row_p, col_p)
        args = (h, S, dinv, x, W0[i], W1[i], cheb_b[i].reshape(1, HH),
                mlp_bn1_g[i].reshape(1, HH), mlp_bn1_b[i].reshape(1, HH),
                mlp_W[i], mlp_b[i].reshape(1, HH),
                mlp_bn2_g[i].reshape(1, HH), mlp_bn2_b[i].reshape(1, HH))
        if i < 2:
            x, h, g = _post_fused_call(*args, bn_gamma[i + 1].reshape(1, HH),
                                       bn_beta[i + 1].reshape(1, HH))
        else:
            x = _post_final_call(*args)
    return x
